# baseline (speedup 1.0000x reference)
import functools

import jax
import jax.numpy as jnp
from jax.experimental import pallas as pl
from jax.experimental.pallas import tpu as pltpu

H = 32
NGATES = 4
GATES = ["i", "f", "c", "o"]
SMALL = ["agent_summ", "hideout_summ", "state_summ", "hideout", "timestep"]
BLK = {nt: i for i, nt in enumerate(SMALL)}
SW = H * len(SMALL)
SMALL_EDGES = [
    ("hideout", "hideout_summ"),
    ("hideout_summ", "state_summ"),
    ("agent_summ", "state_summ"),
    ("timestep", "state_summ"),
    ("hideout_summ", "hideout"),
    ("state_summ", "hideout_summ"),
    ("state_summ", "agent_summ"),
    ("state_summ", "timestep"),
]


def _ek(s, d):
    return s + "__" + d


def _pack_weights(params):
    WxT = jnp.concatenate(
        [params[g]["W"]["agent"] for g in GATES], axis=1).T
    WrT = jnp.concatenate(
        [params[g]["conv"][_ek("agent_summ", "agent")]["Wr"] for g in GATES],
        axis=1).T
    WlgT = jnp.concatenate(
        [params[g]["conv"][_ek("agent_summ", "agent")]["Wl"] for g in GATES],
        axis=1).T
    gb0T = jnp.concatenate(
        [
            (
                params[g]["conv"][_ek("agent_summ", "agent")]["bl"][None, :]
                + params[g]["b"]["agent"]
            )
            for g in GATES
        ],
        axis=1,
    ).T

    ZH = jnp.zeros((H, H), jnp.float32)
    rows = []
    for s in SMALL:
        row = []
        for g in GATES:
            conv = params[g]["conv"]
            for d in SMALL:
                acc = ZH
                for es, ed in SMALL_EDGES:
                    if ed != d:
                        continue
                    p = conv[_ek(es, ed)]
                    if es == s:
                        acc = acc + p["Wl"]
                    if ed == s:
                        acc = acc + p["Wr"]
                if d == "agent_summ" and s == "agent_summ":
                    acc = acc + conv[_ek("agent", "agent_summ")]["Wr"]
                row.append(acc)
        rows.append(jnp.concatenate(row, axis=1))
    Mb0 = jnp.concatenate(rows, axis=0)

    Z2 = jnp.zeros((2, H), jnp.float32)
    Z1 = jnp.zeros((1, H), jnp.float32)
    mbm, mbx, bxt_r, brow_r = [], [], [], []
    for g in GATES:
        conv = params[g]["conv"]
        for d in SMALL:
            mbm.append(conv[_ek("agent", "agent_summ")]["Wl"]
                       if d == "agent_summ" else ZH)
            mbx.append(params[g]["W"]["hideout"] if d == "hideout" else Z2)
            bxt_r.append(params[g]["W"]["timestep"]
                         if d == "timestep" else Z1)
            b = params[g]["b"][d]
            for es, ed in SMALL_EDGES:
                if ed == d:
                    b = b + conv[_ek(es, ed)]["bl"][None, :]
            if d == "agent_summ":
                b = b + conv[_ek("agent", "agent_summ")]["bl"][None, :]
            brow_r.append(b)
    Mbm = jnp.concatenate(mbm, axis=1)
    Mbx = jnp.concatenate(mbx, axis=1)
    bxt = jnp.concatenate(bxt_r, axis=1)
    brow = jnp.concatenate(brow_r, axis=1)

    sc_a = jnp.repeat(jnp.array([0.5, 0.5, 1.0, 0.5], jnp.float32),
                      H)[:, None]
    sc_s = jnp.repeat(jnp.array([0.5, 0.5, 1.0, 0.5], jnp.float32),
                      SW)[:, None]
    return (WxT * sc_a, WrT * sc_a, WlgT * sc_a, gb0T * sc_a,
            Mb0.T * sc_s, Mbm.T * sc_s, Mbx.T * sc_s, bxt.T * sc_s,
            brow.T * sc_s)


def _dot(a, b):
    return jnp.dot(a, b, preferred_element_type=jnp.float32)


def _sig(z):
    return 0.5 + 0.5 * jnp.tanh(z)


BF = jnp.bfloat16


def _body(x_ref, abr_ref, xh_ref, xt_ref, Wx_ref, Wr_ref, Wlg_ref,
          gb0_ref, Mb0_ref, Mbm_ref, Mbx_ref, bxt_ref, brow_ref, out_ref,
          fi_a, c_a, h_s, c_s, m_s, wall_s, *, TILE, NTILES, TA, SEQ, NB):
    t = pl.program_id(0)
    j = pl.program_id(1)
    cols = pl.ds(j * TILE, TILE)

    @pl.when(t == 0)
    def _init_tile():
        ab_r = abr_ref[0]
        oT = (jax.lax.broadcasted_iota(jnp.int32, (NB, TILE), 0) == ab_r
              ).astype(BF)
        fi_a[8:, cols] = jnp.concatenate(
            [jnp.zeros((H, TILE), BF), jnp.ones((8, TILE), BF), oT], axis=0)
        c_a[:, cols] = jnp.zeros((H, TILE), jnp.float32)

    @pl.when((t == 0) & (j == 0))
    def _zero_small():
        h_s[...] = jnp.zeros((SW, NB), jnp.float32)
        c_s[...] = jnp.zeros((SW, NB), jnp.float32)

    @pl.when(j == 0)
    def _start_step():
        gb = (_dot(Wlg_ref[...], h_s[0:H, :]) + gb0_ref[...]).astype(BF)
        wall_s[...] = jnp.concatenate(
            [Wx_ref[...], Wr_ref[...], jnp.zeros((NGATES * H, 8), BF), gb],
            axis=1)
        m_s[...] = jnp.zeros((H + 8, NB), jnp.float32)

    fi_a[0:8, cols] = x_ref[0]

    h1_prev = fi_a[8:48, cols]
    oT = fi_a[48:, cols]
    c_prev = c_a[:, cols]
    m_s[...] += jax.lax.dot_general(
        h1_prev, oT, (((1,), (1,)), ((), ())),
        preferred_element_type=jnp.float32)

    pre = _dot(wall_s[...], fi_a[:, cols])
    ig = _sig(pre[0:H, :])
    fg = _sig(pre[H:2 * H, :])
    gg = jnp.tanh(pre[2 * H:3 * H, :])
    og = _sig(pre[3 * H:4 * H, :])
    c_new = fg * c_prev + ig * gg
    h_new = og * jnp.tanh(c_new)
    valid = (j * TILE + jax.lax.broadcasted_iota(jnp.int32, (1, TILE), 1)) < TA
    fi_a[8:40, cols] = jnp.where(valid, h_new, 0.0).astype(BF)
    c_a[:, cols] = jnp.where(valid, c_new, 0.0)

    @pl.when(j == NTILES - 1)
    def _small_step():
        m = m_s[0:H, :] / jnp.maximum(m_s[H:H + 1, :], 1.0)
        pre_s = (_dot(Mb0_ref[...], h_s[...]) + _dot(Mbm_ref[...], m)
                 + _dot(Mbx_ref[...], xh_ref[...])
                 + bxt_ref[...] * xt_ref[0]
                 + brow_ref[...])
        i_s = _sig(pre_s[0:SW, :])
        f_s = _sig(pre_s[SW:2 * SW, :])
        g_s = jnp.tanh(pre_s[2 * SW:3 * SW, :])
        o_s = _sig(pre_s[3 * SW:4 * SW, :])
        c_ns = f_s * c_s[...] + i_s * g_s
        h_ns = o_s * jnp.tanh(c_ns)
        c_s[...] = c_ns
        h_s[...] = h_ns

        @pl.when(t == SEQ - 1)
        def _emit():
            out_ref[...] = h_ns[2 * H:3 * H, :]


def kernel(agent_feats, hideout_obs, timestep_obs, params, agent_batch):
    SEQ, TA, F = agent_feats.shape
    NB = hideout_obs.shape[0]
    TILE = 4096
    NTILES = max(1, -(-TA // TILE))
    TAP = NTILES * TILE

    WxT, WrT, WlgT, gb0T, Mb0T, MbmT, MbxT, bxtT, browT = _pack_weights(params)

    ab = agent_batch.astype(jnp.int32)
    abp = jnp.pad(ab, (0, TAP - TA), constant_values=127)
    ab_row = abp.reshape(NTILES, 1, TILE)
    afT = agent_feats.transpose(0, 2, 1).astype(BF)
    xhT = hideout_obs.T
    xt3 = timestep_obs.T.reshape(SEQ, 1, NB)

    body = functools.partial(_body, TILE=TILE, NTILES=NTILES, TA=TA, SEQ=SEQ,
                             NB=NB)
    grid = (SEQ, NTILES)
    outT = pl.pallas_call(
        body,
        grid=grid,
        in_specs=[
            pl.BlockSpec((1, F, TILE), lambda t, j: (t, 0, j)),
            pl.BlockSpec((1, 1, TILE), lambda t, j: (j, 0, 0)),
            pl.BlockSpec((2, NB), lambda t, j: (0, 0)),
            pl.BlockSpec((1, 1, NB), lambda t, j: (t, 0, 0)),
            pl.BlockSpec((NGATES * H, F), lambda t, j: (0, 0)),
            pl.BlockSpec((NGATES * H, H), lambda t, j: (0, 0)),
            pl.BlockSpec((NGATES * H, H), lambda t, j: (0, 0)),
            pl.BlockSpec((NGATES * H, 1), lambda t, j: (0, 0)),
            pl.BlockSpec((NGATES * SW, SW), lambda t, j: (0, 0)),
            pl.BlockSpec((NGATES * SW, H), lambda t, j: (0, 0)),
            pl.BlockSpec((NGATES * SW, 2), lambda t, j: (0, 0)),
            pl.BlockSpec((NGATES * SW, 1), lambda t, j: (0, 0)),
            pl.BlockSpec((NGATES * SW, 1), lambda t, j: (0, 0)),
        ],
        out_specs=pl.BlockSpec((H, NB), lambda t, j: (0, 0)),
        out_shape=jax.ShapeDtypeStruct((H, NB), jnp.float32),
        scratch_shapes=[
            pltpu.VMEM((112, TAP), BF),
            pltpu.VMEM((H, TAP), jnp.float32),
            pltpu.VMEM((SW, NB), jnp.float32),
            pltpu.VMEM((SW, NB), jnp.float32),
            pltpu.VMEM((H + 8, NB), jnp.float32),
            pltpu.VMEM((NGATES * H, 112), BF),
        ],
    )(afT, ab_row, xhT, xt3, WxT.astype(BF), WrT.astype(BF), WlgT,
      gb0T, Mb0T, MbmT, MbxT, bxtT, browT)
    return outT.T


# speedup vs baseline: 1.0891x; 1.0891x over previous
import functools

import jax
import jax.numpy as jnp
from jax.experimental import pallas as pl
from jax.experimental.pallas import tpu as pltpu

H = 32
NGATES = 4
GATES = ["i", "f", "c", "o"]
SMALL = ["agent_summ", "hideout_summ", "state_summ", "hideout", "timestep"]
BLK = {nt: i for i, nt in enumerate(SMALL)}
SW = H * len(SMALL)
SMALL_EDGES = [
    ("hideout", "hideout_summ"),
    ("hideout_summ", "state_summ"),
    ("agent_summ", "state_summ"),
    ("timestep", "state_summ"),
    ("hideout_summ", "hideout"),
    ("state_summ", "hideout_summ"),
    ("state_summ", "agent_summ"),
    ("state_summ", "timestep"),
]


def _ek(s, d):
    return s + "__" + d


def _pack_weights(params):
    WxT = jnp.concatenate(
        [params[g]["W"]["agent"] for g in GATES], axis=1).T
    WrT = jnp.concatenate(
        [params[g]["conv"][_ek("agent_summ", "agent")]["Wr"] for g in GATES],
        axis=1).T
    WlgT = jnp.concatenate(
        [params[g]["conv"][_ek("agent_summ", "agent")]["Wl"] for g in GATES],
        axis=1).T
    gb0T = jnp.concatenate(
        [
            (
                params[g]["conv"][_ek("agent_summ", "agent")]["bl"][None, :]
                + params[g]["b"]["agent"]
            )
            for g in GATES
        ],
        axis=1,
    ).T

    ZH = jnp.zeros((H, H), jnp.float32)
    rows = []
    for s in SMALL:
        row = []
        for g in GATES:
            conv = params[g]["conv"]
            for d in SMALL:
                acc = ZH
                for es, ed in SMALL_EDGES:
                    if ed != d:
                        continue
                    p = conv[_ek(es, ed)]
                    if es == s:
                        acc = acc + p["Wl"]
                    if ed == s:
                        acc = acc + p["Wr"]
                if d == "agent_summ" and s == "agent_summ":
                    acc = acc + conv[_ek("agent", "agent_summ")]["Wr"]
                row.append(acc)
        rows.append(jnp.concatenate(row, axis=1))
    Mb0 = jnp.concatenate(rows, axis=0)

    Z2 = jnp.zeros((2, H), jnp.float32)
    Z1 = jnp.zeros((1, H), jnp.float32)
    mbm, mbx, bxt_r, brow_r = [], [], [], []
    for g in GATES:
        conv = params[g]["conv"]
        for d in SMALL:
            mbm.append(conv[_ek("agent", "agent_summ")]["Wl"]
                       if d == "agent_summ" else ZH)
            mbx.append(params[g]["W"]["hideout"] if d == "hideout" else Z2)
            bxt_r.append(params[g]["W"]["timestep"]
                         if d == "timestep" else Z1)
            b = params[g]["b"][d]
            for es, ed in SMALL_EDGES:
                if ed == d:
                    b = b + conv[_ek(es, ed)]["bl"][None, :]
            if d == "agent_summ":
                b = b + conv[_ek("agent", "agent_summ")]["bl"][None, :]
            brow_r.append(b)
    Mbm = jnp.concatenate(mbm, axis=1)
    Mbx = jnp.concatenate(mbx, axis=1)
    bxt = jnp.concatenate(bxt_r, axis=1)
    brow = jnp.concatenate(brow_r, axis=1)

    sc_a = jnp.repeat(jnp.array([0.5, 0.5, 1.0, 0.5], jnp.float32),
                      H)[:, None]
    sc_s = jnp.repeat(jnp.array([0.5, 0.5, 1.0, 0.5], jnp.float32),
                      SW)[:, None]
    return (WxT * sc_a, WrT * sc_a, WlgT * sc_a, gb0T * sc_a,
            Mb0.T * sc_s, Mbm.T * sc_s, Mbx.T * sc_s, bxt.T * sc_s,
            brow.T * sc_s)


def _dot(a, b):
    return jnp.dot(a, b, preferred_element_type=jnp.float32)


def _sig(z):
    return 0.5 + 0.5 * jnp.tanh(z)


BF = jnp.bfloat16


def _body(x_ref, abr_ref, xh_ref, xt_ref, Wx_ref, Wr_ref, Wlg_ref,
          gb0_ref, Mb0_ref, Mbm_ref, Mbx_ref, bxt_ref, brow_ref, out_ref,
          h_a, c_a, h_s, c_s, m_s, wall_s, *, TILE, NTILES, TA, SEQ, NB):
    t = pl.program_id(0)
    j = pl.program_id(1)
    cols = pl.ds(j * TILE, TILE)

    @pl.when(t == 0)
    def _zero_tile():
        h_a[:, cols] = jnp.concatenate(
            [jnp.zeros((H, TILE), BF), jnp.ones((8, TILE), BF)], axis=0)
        c_a[:, cols] = jnp.zeros((H, TILE), jnp.float32)

    @pl.when((t == 0) & (j == 0))
    def _zero_small():
        h_s[...] = jnp.zeros((SW, NB), jnp.float32)
        c_s[...] = jnp.zeros((SW, NB), jnp.float32)

    @pl.when(j == 0)
    def _start_step():
        gb = (_dot(Wlg_ref[...], h_s[0:H, :]) + gb0_ref[...]).astype(BF)
        wall_s[...] = jnp.concatenate(
            [Wx_ref[...], Wr_ref[...], jnp.zeros((NGATES * H, 8), BF), gb],
            axis=1)
        m_s[...] = jnp.zeros((H + 8, NB), jnp.float32)

    ab_r = abr_ref[0]
    oT = (jax.lax.broadcasted_iota(jnp.int32, (NB, TILE), 0) == ab_r
          ).astype(BF)

    h1_prev = h_a[:, cols]
    c_prev = c_a[:, cols]
    m_s[...] += jax.lax.dot_general(
        h1_prev, oT, (((1,), (1,)), ((), ())),
        preferred_element_type=jnp.float32)

    x = x_ref[0]
    fused_in = jnp.concatenate([x, h1_prev, oT], axis=0)
    pre = _dot(wall_s[...], fused_in)
    ig = _sig(pre[0:H, :])
    fg = _sig(pre[H:2 * H, :])
    gg = jnp.tanh(pre[2 * H:3 * H, :])
    og = _sig(pre[3 * H:4 * H, :])
    c_new = fg * c_prev + ig * gg
    h_new = og * jnp.tanh(c_new)
    valid = (j * TILE + jax.lax.broadcasted_iota(jnp.int32, (1, TILE), 1)) < TA
    h_a[0:H, cols] = jnp.where(valid, h_new, 0.0).astype(BF)
    c_a[:, cols] = jnp.where(valid, c_new, 0.0)

    @pl.when(j == NTILES - 1)
    def _small_step():
        m = m_s[0:H, :] / jnp.maximum(m_s[H:H + 1, :], 1.0)
        pre_s = (_dot(Mb0_ref[...], h_s[...]) + _dot(Mbm_ref[...], m)
                 + _dot(Mbx_ref[...], xh_ref[...])
                 + bxt_ref[...] * xt_ref[0]
                 + brow_ref[...])
        i_s = _sig(pre_s[0:SW, :])
        f_s = _sig(pre_s[SW:2 * SW, :])
        g_s = jnp.tanh(pre_s[2 * SW:3 * SW, :])
        o_s = _sig(pre_s[3 * SW:4 * SW, :])
        c_ns = f_s * c_s[...] + i_s * g_s
        h_ns = o_s * jnp.tanh(c_ns)
        c_s[...] = c_ns
        h_s[...] = h_ns

        @pl.when(t == SEQ - 1)
        def _emit():
            out_ref[...] = h_ns[2 * H:3 * H, :]


def kernel(agent_feats, hideout_obs, timestep_obs, params, agent_batch):
    SEQ, TA, F = agent_feats.shape
    NB = hideout_obs.shape[0]
    TILE = 8192
    NTILES = max(1, -(-TA // TILE))
    TAP = NTILES * TILE

    WxT, WrT, WlgT, gb0T, Mb0T, MbmT, MbxT, bxtT, browT = _pack_weights(params)

    ab = agent_batch.astype(jnp.int32)
    abp = jnp.pad(ab, (0, TAP - TA), constant_values=127)
    ab_row = abp.reshape(NTILES, 1, TILE)
    afT = agent_feats.transpose(0, 2, 1).astype(BF)
    xhT = hideout_obs.T
    xt3 = timestep_obs.T.reshape(SEQ, 1, NB)

    body = functools.partial(_body, TILE=TILE, NTILES=NTILES, TA=TA, SEQ=SEQ,
                             NB=NB)
    grid = (SEQ, NTILES)
    outT = pl.pallas_call(
        body,
        grid=grid,
        in_specs=[
            pl.BlockSpec((1, F, TILE), lambda t, j: (t, 0, j)),
            pl.BlockSpec((1, 1, TILE), lambda t, j: (j, 0, 0)),
            pl.BlockSpec((2, NB), lambda t, j: (0, 0)),
            pl.BlockSpec((1, 1, NB), lambda t, j: (t, 0, 0)),
            pl.BlockSpec((NGATES * H, F), lambda t, j: (0, 0)),
            pl.BlockSpec((NGATES * H, H), lambda t, j: (0, 0)),
            pl.BlockSpec((NGATES * H, H), lambda t, j: (0, 0)),
            pl.BlockSpec((NGATES * H, 1), lambda t, j: (0, 0)),
            pl.BlockSpec((NGATES * SW, SW), lambda t, j: (0, 0)),
            pl.BlockSpec((NGATES * SW, H), lambda t, j: (0, 0)),
            pl.BlockSpec((NGATES * SW, 2), lambda t, j: (0, 0)),
            pl.BlockSpec((NGATES * SW, 1), lambda t, j: (0, 0)),
            pl.BlockSpec((NGATES * SW, 1), lambda t, j: (0, 0)),
        ],
        out_specs=pl.BlockSpec((H, NB), lambda t, j: (0, 0)),
        out_shape=jax.ShapeDtypeStruct((H, NB), jnp.float32),
        scratch_shapes=[
            pltpu.VMEM((H + 8, TAP), BF),
            pltpu.VMEM((H, TAP), jnp.float32),
            pltpu.VMEM((SW, NB), jnp.float32),
            pltpu.VMEM((SW, NB), jnp.float32),
            pltpu.VMEM((H + 8, NB), jnp.float32),
            pltpu.VMEM((NGATES * H, 112), BF),
        ],
    )(afT, ab_row, xhT, xt3, WxT.astype(BF), WrT.astype(BF), WlgT,
      gb0T, Mb0T, MbmT, MbxT, bxtT, browT)
    return outT.T
